# Initial kernel scaffold; baseline (speedup 1.0000x reference)
import jax
import jax.numpy as jnp
from jax.experimental import pallas as pl


def kernel(edge_attr, W0, W1, W2):
    raise NotImplementedError("write your pallas kernel here")


# speedup vs baseline: 8.1841x; 8.1841x over previous
import jax
import jax.numpy as jnp
from jax.experimental import pallas as pl

EMB = 128
BE = 4000


def _body(attr_ref, w0_ref, w1_ref, w2_ref, out_ref):
    a = attr_ref[...]

    def emb(col, w):
        idx = a[:, col:col + 1]
        oh = (idx == jax.lax.broadcasted_iota(jnp.int32, (1, 8), 1)
              ).astype(jnp.float32)
        return jnp.dot(oh, w, preferred_element_type=jnp.float32)

    out_ref[...] = emb(0, w0_ref[...]) + emb(1, w1_ref[...]) + emb(2, w2_ref[...])


def kernel(edge_attr, W0, W1, W2):
    E = edge_attr.shape[0]
    attr = edge_attr.astype(jnp.int32)

    def pad8(w):
        return jnp.zeros((8, EMB), jnp.float32).at[:w.shape[0]].set(w)

    grid = (E // BE,)
    return pl.pallas_call(
        _body,
        grid=grid,
        in_specs=[
            pl.BlockSpec((BE, 3), lambda i: (i, 0)),
            pl.BlockSpec((8, EMB), lambda i: (0, 0)),
            pl.BlockSpec((8, EMB), lambda i: (0, 0)),
            pl.BlockSpec((8, EMB), lambda i: (0, 0)),
        ],
        out_specs=pl.BlockSpec((BE, EMB), lambda i: (i, 0)),
        out_shape=jax.ShapeDtypeStruct((E, EMB), jnp.float32),
    )(attr, pad8(W0), pad8(W1), pad8(W2))
